# baseline (speedup 1.0000x reference)
import jax
import jax.numpy as jnp
from jax.experimental import pallas as pl

_BM = 400


def _main_kernel(adj_ref, xb_ref, xrow_ref, w1t_ref, w2t_ref, out_ref):
    a = adj_ref[...].astype(jnp.bfloat16)
    agg = jnp.dot(a, xb_ref[...], preferred_element_type=jnp.float32)
    proj = jnp.dot(xrow_ref[...], w1t_ref[...],
                   preferred_element_type=jnp.float32)
    proj += jnp.dot(agg, w2t_ref[...], preferred_element_type=jnp.float32)
    out_ref[...] = proj


def _bn_kernel(o_ref, g_ref, b_ref, out_ref):
    o = o_ref[...]
    mean = jnp.mean(o, axis=0, keepdims=True)
    c = o - mean
    var = jnp.mean(c * c, axis=0, keepdims=True)
    scale = g_ref[...] * jax.lax.rsqrt(var + 1e-5)
    out_ref[...] = c * scale + b_ref[...]


def kernel(x, adj, W, gamma, beta):
    n, d_in = x.shape
    d_out = W.shape[0]
    w1t = W[:, :d_in].T
    w2t = W[:, d_in:].T
    xb = x.astype(jnp.bfloat16)

    out_pre = pl.pallas_call(
        _main_kernel,
        grid=(n // _BM,),
        in_specs=[
            pl.BlockSpec((_BM, n), lambda i: (i, 0)),
            pl.BlockSpec((n, d_in), lambda i: (0, 0)),
            pl.BlockSpec((_BM, d_in), lambda i: (i, 0)),
            pl.BlockSpec((d_in, d_out), lambda i: (0, 0)),
            pl.BlockSpec((d_in, d_out), lambda i: (0, 0)),
        ],
        out_specs=pl.BlockSpec((_BM, d_out), lambda i: (i, 0)),
        out_shape=jax.ShapeDtypeStruct((n, d_out), jnp.float32),
    )(adj, xb, x, w1t, w2t)

    out = pl.pallas_call(
        _bn_kernel,
        out_shape=jax.ShapeDtypeStruct((n, d_out), jnp.float32),
    )(out_pre, gamma.reshape(1, d_out), beta.reshape(1, d_out))
    return out


# speedup vs baseline: 1.0241x; 1.0241x over previous
import jax
import jax.numpy as jnp
from jax.experimental import pallas as pl

_BM = 200


def _main_kernel(adja_ref, adjb_ref, xb_ref, xrow_ref,
                 w1t_ref, w2t_ref, out_ref):
    xb = xb_ref[...]
    w1t = w1t_ref[...]
    w2t = w2t_ref[...]
    a = adja_ref[...].astype(jnp.bfloat16)
    agg = jnp.dot(a, xb, preferred_element_type=jnp.float32)
    proj = jnp.dot(xrow_ref[:_BM], w1t, preferred_element_type=jnp.float32)
    proj += jnp.dot(agg, w2t, preferred_element_type=jnp.float32)
    out_ref[:_BM] = proj
    b = adjb_ref[...].astype(jnp.bfloat16)
    aggb = jnp.dot(b, xb, preferred_element_type=jnp.float32)
    projb = jnp.dot(xrow_ref[_BM:], w1t, preferred_element_type=jnp.float32)
    projb += jnp.dot(aggb, w2t, preferred_element_type=jnp.float32)
    out_ref[_BM:] = projb


def _bn_kernel(o_ref, g_ref, b_ref, out_ref):
    o = o_ref[...]
    mean = jnp.mean(o, axis=0, keepdims=True)
    c = o - mean
    var = jnp.mean(c * c, axis=0, keepdims=True)
    scale = g_ref[...] * jax.lax.rsqrt(var + 1e-5)
    out_ref[...] = c * scale + b_ref[...]


def kernel(x, adj, W, gamma, beta):
    n, d_in = x.shape
    d_out = W.shape[0]
    w1t = W[:, :d_in].T
    w2t = W[:, d_in:].T
    xb = x.astype(jnp.bfloat16)

    out_pre = pl.pallas_call(
        _main_kernel,
        grid=(n // (2 * _BM),),
        in_specs=[
            pl.BlockSpec((_BM, n), lambda i: (2 * i, 0)),
            pl.BlockSpec((_BM, n), lambda i: (2 * i + 1, 0)),
            pl.BlockSpec((n, d_in), lambda i: (0, 0)),
            pl.BlockSpec((2 * _BM, d_in), lambda i: (i, 0)),
            pl.BlockSpec((d_in, d_out), lambda i: (0, 0)),
            pl.BlockSpec((d_in, d_out), lambda i: (0, 0)),
        ],
        out_specs=pl.BlockSpec((2 * _BM, d_out), lambda i: (i, 0)),
        out_shape=jax.ShapeDtypeStruct((n, d_out), jnp.float32),
    )(adj, adj, xb, x, w1t, w2t)

    out = pl.pallas_call(
        _bn_kernel,
        out_shape=jax.ShapeDtypeStruct((n, d_out), jnp.float32),
    )(out_pre, gamma.reshape(1, d_out), beta.reshape(1, d_out))
    return out


# speedup vs baseline: 1.0719x; 1.0467x over previous
import jax
import jax.numpy as jnp
from jax.experimental import pallas as pl
from jax.experimental.pallas import tpu as pltpu

_BM = 200


def _main_kernel(adja_ref, adjb_ref, xb_ref, xrow_ref, w1t_ref, w2t_ref,
                 g_ref, b_ref, out_ref, sum_ref, sq_ref):
    i = pl.program_id(0)
    nsteps = pl.num_programs(0)

    @pl.when(i == 0)
    def _init():
        sum_ref[...] = jnp.zeros_like(sum_ref)
        sq_ref[...] = jnp.zeros_like(sq_ref)

    xb = xb_ref[...]
    w1t = w1t_ref[...]
    w2t = w2t_ref[...]
    a = adja_ref[...].astype(jnp.bfloat16)
    agg = jnp.dot(a, xb, preferred_element_type=jnp.float32)
    proj = jnp.dot(xrow_ref[:_BM], w1t, preferred_element_type=jnp.float32)
    proj += jnp.dot(agg, w2t, preferred_element_type=jnp.float32)
    b = adjb_ref[...].astype(jnp.bfloat16)
    aggb = jnp.dot(b, xb, preferred_element_type=jnp.float32)
    projb = jnp.dot(xrow_ref[_BM:], w1t, preferred_element_type=jnp.float32)
    projb += jnp.dot(aggb, w2t, preferred_element_type=jnp.float32)

    row0 = i * 2 * _BM
    out_ref[pl.ds(row0, _BM)] = proj
    out_ref[pl.ds(row0 + _BM, _BM)] = projb
    sum_ref[...] += (jnp.sum(proj, axis=0, keepdims=True)
                     + jnp.sum(projb, axis=0, keepdims=True))
    sq_ref[...] += (jnp.sum(proj * proj, axis=0, keepdims=True)
                    + jnp.sum(projb * projb, axis=0, keepdims=True))

    @pl.when(i == nsteps - 1)
    def _finalize():
        inv_n = 1.0 / out_ref.shape[0]
        mean = sum_ref[...] * inv_n
        var = sq_ref[...] * inv_n - mean * mean
        scale = g_ref[...] * jax.lax.rsqrt(var + 1e-5)
        shift = b_ref[...] - mean * scale
        out_ref[...] = out_ref[...] * scale + shift


def kernel(x, adj, W, gamma, beta):
    n, d_in = x.shape
    d_out = W.shape[0]
    w1t = W[:, :d_in].T
    w2t = W[:, d_in:].T
    xb = x.astype(jnp.bfloat16)

    return pl.pallas_call(
        _main_kernel,
        grid=(n // (2 * _BM),),
        in_specs=[
            pl.BlockSpec((_BM, n), lambda i: (2 * i, 0)),
            pl.BlockSpec((_BM, n), lambda i: (2 * i + 1, 0)),
            pl.BlockSpec((n, d_in), lambda i: (0, 0)),
            pl.BlockSpec((2 * _BM, d_in), lambda i: (i, 0)),
            pl.BlockSpec((d_in, d_out), lambda i: (0, 0)),
            pl.BlockSpec((d_in, d_out), lambda i: (0, 0)),
            pl.BlockSpec((1, d_out), lambda i: (0, 0)),
            pl.BlockSpec((1, d_out), lambda i: (0, 0)),
        ],
        out_specs=pl.BlockSpec((n, d_out), lambda i: (0, 0)),
        out_shape=jax.ShapeDtypeStruct((n, d_out), jnp.float32),
        scratch_shapes=[
            pltpu.VMEM((1, d_out), jnp.float32),
            pltpu.VMEM((1, d_out), jnp.float32),
        ],
    )(adj, adj, xb, x, w1t, w2t,
      gamma.reshape(1, d_out), beta.reshape(1, d_out))
